# baseline (speedup 1.0000x reference)
import functools

import numpy as np
import jax
import jax.numpy as jnp
from jax.experimental import pallas as pl
from jax.experimental.pallas import tpu as pltpu


def _bilinear_matrix(out_size, in_size):
    if in_size == 1:
        return jnp.ones((out_size, 1), jnp.float32)
    scale = np.float32((in_size - 1) / (out_size - 1))
    pos = np.arange(out_size, dtype=np.float32) * scale
    low = np.clip(np.floor(pos).astype(np.int64), 0, in_size - 2)
    frac = pos - low.astype(np.float32)
    m = np.zeros((out_size, in_size), np.float32)
    rows = np.arange(out_size)
    m[rows, low] += 1.0 - frac
    m[rows, low + 1] += frac
    return jnp.asarray(m)


def _rows_kernel(h, oh, n_rep, tile_oh, aw_ref, x_ref, o_ref):
    base = pl.program_id(0) * tile_oh
    for j in range(tile_oh):
        i = base + j
        pos = i.astype(jnp.float32) * np.float32((h - 1) / (oh - 1))
        low = jnp.minimum(jnp.floor(pos).astype(jnp.int32), h - 2)
        frac = pos - low.astype(jnp.float32)
        x0 = x_ref[low]
        x1 = x_ref[low + 1]
        tmp = (1.0 - frac) * x0 + frac * x1
        out = jnp.dot(aw_ref[...], tmp,
                      preferred_element_type=jnp.float32)
        for c in range(n_rep):
            o_ref[c, j] = out


def kernel(x, out_hw=(224, 224)):
    assert x.ndim == 4, "expected NCHW input"
    n, c_in, h, w = x.shape
    oh, ow = out_hw
    c_out = c_in if c_in >= 3 else 3

    if c_in == 1 and h > 1:
        a_w = _bilinear_matrix(ow, w)
        xt = jnp.transpose(x, (1, 2, 3, 0)).reshape(h, w, n)
        tile_oh = 16 if oh % 16 == 0 else 1
        out_t = pl.pallas_call(
            functools.partial(_rows_kernel, h, oh, c_out, tile_oh),
            out_shape=jax.ShapeDtypeStruct((c_out, oh, ow, n), x.dtype),
            grid_spec=pltpu.PrefetchScalarGridSpec(
                num_scalar_prefetch=0,
                grid=(oh // tile_oh,),
                in_specs=[
                    pl.BlockSpec((ow, w), lambda i: (0, 0)),
                    pl.BlockSpec((h, w, n), lambda i: (0, 0, 0)),
                ],
                out_specs=pl.BlockSpec((c_out, tile_oh, ow, n),
                                       lambda i: (0, i, 0, 0)),
            ),
            compiler_params=pltpu.CompilerParams(
                dimension_semantics=("parallel",)),
            cost_estimate=pl.CostEstimate(
                flops=2 * n * (oh * w + oh * ow * w),
                transcendentals=0,
                bytes_accessed=(n * h * w + n * c_out * oh * ow)
                * x.dtype.itemsize,
            ),
        )(a_w, xt)
        return jnp.transpose(out_t, (3, 0, 1, 2))

    a_h = _bilinear_matrix(oh, h)
    a_wtt = _bilinear_matrix(ow, w).T

    def _general_kernel(ah_ref, x_ref, awt_ref, o_ref):
        xi = x_ref[0, 0]
        tmp = jnp.dot(ah_ref[...], xi, preferred_element_type=jnp.float32)
        o_ref[0, 0] = jnp.dot(tmp, awt_ref[...],
                              preferred_element_type=jnp.float32)

    x_map = (lambda nn_, cc: (nn_, cc, 0, 0)) if c_in >= 3 else (
        lambda nn_, cc: (nn_, cc % c_in, 0, 0))
    return pl.pallas_call(
        _general_kernel,
        out_shape=jax.ShapeDtypeStruct((n, c_out, oh, ow), x.dtype),
        grid_spec=pltpu.PrefetchScalarGridSpec(
            num_scalar_prefetch=0,
            grid=(n, c_out),
            in_specs=[
                pl.BlockSpec((oh, h), lambda nn_, cc: (0, 0)),
                pl.BlockSpec((1, 1, h, w), x_map),
                pl.BlockSpec((w, ow), lambda nn_, cc: (0, 0)),
            ],
            out_specs=pl.BlockSpec((1, 1, oh, ow),
                                   lambda nn_, cc: (nn_, cc, 0, 0)),
        ),
        compiler_params=pltpu.CompilerParams(
            dimension_semantics=("parallel", "parallel")),
    )(a_h, x, a_wtt)
